# Initial kernel scaffold; baseline (speedup 1.0000x reference)
import jax
import jax.numpy as jnp
from jax.experimental import pallas as pl


def kernel(tokens, input_embedding_table):
    raise NotImplementedError("write your pallas kernel here")


# speedup vs baseline: 2.7380x; 2.7380x over previous
import functools

import jax
import jax.numpy as jnp
from jax import lax
from jax.experimental import pallas as pl
from jax.experimental.pallas import tpu as pltpu
from jax.experimental.pallas import tpu_sc as plsc

EMBED = 64
LANES = 16
NC, NS = 2, 16
NW = NC * NS
CHUNK = 128
SCALE = 8.0


@functools.partial(jax.jit, static_argnames=())
def _embed_sc(tokens3, table):
    nw, nchunks, chunk = tokens3.shape
    n = nw * nchunks * chunk
    mesh = plsc.VectorSubcoreMesh(core_axis_name="c", subcore_axis_name="s")

    @functools.partial(
        pl.kernel,
        mesh=mesh,
        compiler_params=pltpu.CompilerParams(use_tc_tiling_on_sc=False),
        out_type=jax.ShapeDtypeStruct((n, EMBED), jnp.float32),
        scratch_types=[
            pltpu.VMEM((nchunks, chunk), jnp.int32),
            pltpu.VMEM((chunk, EMBED), jnp.float32),
            pltpu.SemaphoreType.DMA,
        ],
    )
    def k(tok_hbm, tab_hbm, out_hbm, idx_v, rows_v, sem):
        wid = lax.axis_index("s") * NC + lax.axis_index("c")
        base = wid * (nchunks * chunk)
        pltpu.sync_copy(tok_hbm.at[wid], idx_v)

        def chunk_body(j, carry):
            pltpu.async_copy(tab_hbm.at[idx_v.at[j]], rows_v, sem).wait()

            def scale_body(r, c):
                for kk in range(EMBED // LANES):
                    sl = pl.ds(kk * LANES, LANES)
                    rows_v[r, sl] = rows_v[r, sl] * SCALE
                return c

            lax.fori_loop(0, chunk, scale_body, 0, unroll=2)
            pltpu.sync_copy(rows_v, out_hbm.at[pl.ds(base + j * chunk, chunk)])
            return carry

        lax.fori_loop(0, nchunks, chunk_body, 0)

    return k(tokens3, table)


def kernel(tokens, input_embedding_table):
    b, l = tokens.shape
    n = b * l
    tokens3 = tokens.reshape(NW, n // (NW * CHUNK), CHUNK).astype(jnp.int32)
    out = _embed_sc(tokens3, input_embedding_table)
    return out.reshape(b, l, EMBED)
